# baseline (speedup 1.0000x reference)
import functools

import jax
import jax.numpy as jnp
from jax import lax
from jax.experimental import pallas as pl
from jax.experimental.pallas import tpu as pltpu
from jax.experimental.pallas import tpu_sc as plsc

B = 16384
V = 1000000
E = 64
CTX = 5
H = 256
OUT = 50

N = B * CTX
P = 2 * E
VP = V // 2


@functools.lru_cache(maxsize=1)
def _make_sc_gather():
    info = plsc.get_sparse_core_info()
    NC, NS = info.num_cores, info.num_subcores
    NW = NC * NS
    n_per_w = N // NW
    CH = 320
    n_ch = n_per_w // CH

    mesh = plsc.VectorSubcoreMesh(core_axis_name="c", subcore_axis_name="s")

    @functools.partial(
        pl.kernel,
        mesh=mesh,
        out_type=jax.ShapeDtypeStruct((N, P), jnp.float32),
        scratch_types=[
            pltpu.VMEM((n_per_w,), jnp.int32),
            pltpu.VMEM((CH, P), jnp.float32),
            pltpu.VMEM((CH, P), jnp.float32),
            pltpu.SemaphoreType.DMA,
            pltpu.SemaphoreType.DMA,
        ],
    )
    def gather_k(table_hbm, idx_hbm, out_hbm, idx_v, buf0, buf1, sem0, sem1):
        wid = lax.axis_index("s") * NC + lax.axis_index("c")
        base = wid * n_per_w
        pltpu.sync_copy(idx_hbm.at[pl.ds(base, n_per_w)], idx_v)
        bufs = (buf0, buf1)
        sems = (sem0, sem1)
        copies = [None, None]
        for ci in range(n_ch):
            s = ci % 2
            copies[s] = pltpu.async_copy(
                table_hbm.at[idx_v.at[pl.ds(ci * CH, CH)]], bufs[s], sems[s]
            )
            if ci > 0:
                p = (ci - 1) % 2
                copies[p].wait()
                pltpu.sync_copy(bufs[p], out_hbm.at[pl.ds(base + (ci - 1) * CH, CH)])
        last = (n_ch - 1) % 2
        copies[last].wait()
        pltpu.sync_copy(bufs[last], out_hbm.at[pl.ds(base + (n_ch - 1) * CH, CH)])

    return gather_k


_BLK = 2048


def _mlp_body(flat_ref, x_ref, w1d_ref, b1_ref, w2_ref, b2_ref, out_ref):
    x = x_ref[...]
    b = (x & 1).astype(jnp.float32)
    nz = (x != 0).astype(jnp.float32)
    w_even = nz * (1.0 - b)
    w_odd = b
    padb = jnp.concatenate([w_even, w_odd], axis=1)
    ri = lax.broadcasted_iota(jnp.int32, (2 * CTX, CTX * P), 0)
    ci = lax.broadcasted_iota(jnp.int32, (2 * CTX, CTX * P), 1)
    grp = ci // E
    sel = jnp.where(ri < CTX, 2 * ri, 2 * (ri - CTX) + 1)
    S = (grp == sel).astype(jnp.float32)
    mask = jnp.dot(padb, S, preferred_element_type=jnp.float32)
    acc = jnp.dot(
        flat_ref[...] * mask, w1d_ref[...], preferred_element_type=jnp.float32
    )
    h = jnp.tanh(acc + b1_ref[...])
    out_ref[...] = (
        jnp.dot(h, w2_ref[...], preferred_element_type=jnp.float32) + b2_ref[...]
    )


def _mlp(flat, x32, W1dup, b1, W2, b2):
    grid = (B // _BLK,)
    return pl.pallas_call(
        _mlp_body,
        grid=grid,
        in_specs=[
            pl.BlockSpec((_BLK, CTX * P), lambda i: (i, 0)),
            pl.BlockSpec((_BLK, CTX), lambda i: (i, 0)),
            pl.BlockSpec((CTX * P, H), lambda i: (0, 0)),
            pl.BlockSpec((1, H), lambda i: (0, 0)),
            pl.BlockSpec((H, OUT), lambda i: (0, 0)),
            pl.BlockSpec((1, OUT), lambda i: (0, 0)),
        ],
        out_specs=pl.BlockSpec((_BLK, OUT), lambda i: (i, 0)),
        out_shape=jax.ShapeDtypeStruct((B, OUT), jnp.float32),
    )(flat, x32, W1dup, b1, W2, b2)


def kernel(x, table, W1, b1, W2, b2):
    x32 = x.astype(jnp.int32)
    pidx = (x32 >> 1).reshape(-1)
    table2 = table.reshape(VP, P)
    pairs = _make_sc_gather()(table2, pidx)
    flat = pairs.reshape(B, CTX * P)
    w1r = W1.reshape(CTX, E, H)
    W1dup = jnp.concatenate([w1r, w1r], axis=1).reshape(CTX * P, H)
    out = _mlp(flat, x32, W1dup, b1.reshape(1, H), W2, b2.reshape(1, OUT))
    return out


# speedup vs baseline: 1.1123x; 1.1123x over previous
import functools

import jax
import jax.numpy as jnp
from jax import lax
from jax.experimental import pallas as pl
from jax.experimental.pallas import tpu as pltpu
from jax.experimental.pallas import tpu_sc as plsc

B = 16384
V = 1000000
E = 64
CTX = 5
H = 256
OUT = 50

N = B * CTX
P = 2 * E


@functools.lru_cache(maxsize=1)
def _make_sc_gather():
    info = plsc.get_sparse_core_info()
    NC, NS = info.num_cores, info.num_subcores
    NW = NC * NS
    n_per_w = N // NW
    CH = 320
    n_ch = n_per_w // CH

    mesh = plsc.VectorSubcoreMesh(core_axis_name="c", subcore_axis_name="s")

    @functools.partial(
        pl.kernel,
        mesh=mesh,
        out_type=jax.ShapeDtypeStruct((N, P), jnp.float32),
        scratch_types=[
            pltpu.VMEM((n_per_w,), jnp.int32),
            pltpu.VMEM((CH, P), jnp.float32),
            pltpu.VMEM((CH, P), jnp.float32),
            pltpu.SemaphoreType.DMA,
            pltpu.SemaphoreType.DMA,
        ],
    )
    def gather_k(table_hbm, idx_hbm, out_hbm, idx_v, buf0, buf1, sem0, sem1):
        wid = lax.axis_index("s") * NC + lax.axis_index("c")
        base = wid * n_per_w
        pltpu.sync_copy(idx_hbm.at[pl.ds(base, n_per_w)], idx_v)
        bufs = (buf0, buf1)
        sems = (sem0, sem1)
        copies = [None, None]
        for ci in range(n_ch):
            s = ci % 2
            copies[s] = pltpu.async_copy(
                table_hbm.at[idx_v.at[pl.ds(ci * CH, CH)]], bufs[s], sems[s]
            )
            if ci > 0:
                p = (ci - 1) % 2
                copies[p].wait()
                pltpu.sync_copy(bufs[p], out_hbm.at[pl.ds(base + (ci - 1) * CH, CH)])
        last = (n_ch - 1) % 2
        copies[last].wait()
        pltpu.sync_copy(bufs[last], out_hbm.at[pl.ds(base + (n_ch - 1) * CH, CH)])

    return gather_k


_BLK = 2048


def _mlp_body(flat_ref, x_ref, t0_ref, w1p_ref, b1_ref, w2_ref, b2_ref, out_ref):
    corr = jnp.concatenate(
        [
            -jnp.dot(
                t0_ref[...],
                w1p_ref[c * P : (c + 1) * P, :],
                preferred_element_type=jnp.float32,
            )
            for c in range(CTX)
        ],
        axis=0,
    )
    padm = (x_ref[...] == 0).astype(jnp.float32)
    acc = jnp.dot(flat_ref[...], w1p_ref[...], preferred_element_type=jnp.float32)
    acc = acc + jnp.dot(padm, corr, preferred_element_type=jnp.float32)
    h = jnp.tanh(acc + b1_ref[...])
    out_ref[...] = (
        jnp.dot(h, w2_ref[...], preferred_element_type=jnp.float32) + b2_ref[...]
    )


def _mlp(flat, x32, t0, W1pad, b1, W2, b2):
    grid = (B // _BLK,)
    return pl.pallas_call(
        _mlp_body,
        grid=grid,
        in_specs=[
            pl.BlockSpec((_BLK, CTX * P), lambda i: (i, 0)),
            pl.BlockSpec((_BLK, CTX), lambda i: (i, 0)),
            pl.BlockSpec((1, P), lambda i: (0, 0)),
            pl.BlockSpec((CTX * P, H), lambda i: (0, 0)),
            pl.BlockSpec((1, H), lambda i: (0, 0)),
            pl.BlockSpec((H, OUT), lambda i: (0, 0)),
            pl.BlockSpec((1, OUT), lambda i: (0, 0)),
        ],
        out_specs=pl.BlockSpec((_BLK, OUT), lambda i: (i, 0)),
        out_shape=jax.ShapeDtypeStruct((B, OUT), jnp.float32),
    )(flat, x32, t0, W1pad, b1, W2, b2)


def kernel(x, table, W1, b1, W2, b2):
    x32 = x.astype(jnp.int32)
    idx = x32.reshape(-1)
    tpad = jnp.pad(table, ((0, 0), (0, P - E)))
    rows = _make_sc_gather()(tpad, idx)
    flat = rows.reshape(B, CTX * P)
    W1pad = jnp.pad(W1.reshape(CTX, E, H), ((0, 0), (0, P - E), (0, 0))).reshape(
        CTX * P, H
    )
    out = _mlp(
        flat, x32, tpad[0:1], W1pad, b1.reshape(1, H), W2, b2.reshape(1, OUT)
    )
    return out


# speedup vs baseline: 1.8070x; 1.6246x over previous
import functools

import jax
import jax.numpy as jnp
from jax import lax
from jax.experimental import pallas as pl
from jax.experimental.pallas import tpu as pltpu
from jax.experimental.pallas import tpu_sc as plsc

B = 16384
V = 1000000
E = 64
CTX = 5
H = 256
OUT = 50

N = B * CTX
P = 2 * E


@functools.lru_cache(maxsize=1)
def _make_sc_gather():
    info = plsc.get_sparse_core_info()
    NC, NS = info.num_cores, info.num_subcores
    NW = NC * NS
    n_per_w = N // NW
    CH = 320
    n_ch = n_per_w // CH

    mesh = plsc.VectorSubcoreMesh(core_axis_name="c", subcore_axis_name="s")

    @functools.partial(
        pl.kernel,
        mesh=mesh,
        out_type=jax.ShapeDtypeStruct((N, P), jnp.float32),
        scratch_types=[
            pltpu.VMEM((n_per_w,), jnp.int32),
            pltpu.VMEM((CH, P), jnp.float32),
            pltpu.VMEM((CH, P), jnp.float32),
            pltpu.SemaphoreType.DMA,
            pltpu.SemaphoreType.DMA,
        ],
    )
    def gather_k(table_hbm, idx_hbm, out_hbm, idx_v, buf0, buf1, sem0, sem1):
        wid = lax.axis_index("s") * NC + lax.axis_index("c")
        base = wid * n_per_w
        pltpu.sync_copy(idx_hbm.at[pl.ds(base, n_per_w)], idx_v)
        bufs = (buf0, buf1)
        sems = (sem0, sem1)
        copies = [None, None]
        for ci in range(n_ch):
            s = ci % 2
            copies[s] = pltpu.async_copy(
                table_hbm.at[idx_v.at[pl.ds(ci * CH, CH)]], bufs[s], sems[s]
            )
            if ci > 0:
                p = (ci - 1) % 2
                copies[p].wait()
                pltpu.sync_copy(bufs[p], out_hbm.at[pl.ds(base + (ci - 1) * CH, CH)])
        last = (n_ch - 1) % 2
        copies[last].wait()
        pltpu.sync_copy(bufs[last], out_hbm.at[pl.ds(base + (n_ch - 1) * CH, CH)])

    return gather_k


_TPB = 8192


def _tr_body(t_ref, out_ref):
    tb = t_ref[...]
    ri = lax.broadcasted_iota(jnp.int32, (E, E), 0)
    ci = lax.broadcasted_iota(jnp.int32, (E, E), 1)
    eye = (ri == ci).astype(jnp.float32)
    tt = lax.dot_general(
        tb, eye, (((0,), (0,)), ((), ())), preferred_element_type=jnp.float32
    )
    out_ref[...] = jnp.concatenate(
        [tt, jnp.zeros((_TPB, P - E), jnp.float32)], axis=1
    )


def _transpose_pad(tT):
    grid = ((V + _TPB - 1) // _TPB,)
    return pl.pallas_call(
        _tr_body,
        grid=grid,
        in_specs=[pl.BlockSpec((E, _TPB), lambda i: (0, i))],
        out_specs=pl.BlockSpec((_TPB, P), lambda i: (i, 0)),
        out_shape=jax.ShapeDtypeStruct((V, P), jnp.float32),
    )(tT)


_BLK = 2048


def _mlp_body(flat_ref, x_ref, t0_ref, w1p_ref, b1_ref, w2_ref, b2_ref, out_ref):
    corr = jnp.concatenate(
        [
            -jnp.dot(
                t0_ref[...],
                w1p_ref[c * P : (c + 1) * P, :],
                preferred_element_type=jnp.float32,
            )
            for c in range(CTX)
        ],
        axis=0,
    )
    padm = (x_ref[...] == 0).astype(jnp.float32)
    acc = jnp.dot(flat_ref[...], w1p_ref[...], preferred_element_type=jnp.float32)
    acc = acc + jnp.dot(padm, corr, preferred_element_type=jnp.float32)
    h = jnp.tanh(acc + b1_ref[...])
    out_ref[...] = (
        jnp.dot(h, w2_ref[...], preferred_element_type=jnp.float32) + b2_ref[...]
    )


def _mlp(flat, x32, t0, W1pad, b1, W2, b2):
    grid = (B // _BLK,)
    return pl.pallas_call(
        _mlp_body,
        grid=grid,
        in_specs=[
            pl.BlockSpec((_BLK, CTX * P), lambda i: (i, 0)),
            pl.BlockSpec((_BLK, CTX), lambda i: (i, 0)),
            pl.BlockSpec((1, P), lambda i: (0, 0)),
            pl.BlockSpec((CTX * P, H), lambda i: (0, 0)),
            pl.BlockSpec((1, H), lambda i: (0, 0)),
            pl.BlockSpec((H, OUT), lambda i: (0, 0)),
            pl.BlockSpec((1, OUT), lambda i: (0, 0)),
        ],
        out_specs=pl.BlockSpec((_BLK, OUT), lambda i: (i, 0)),
        out_shape=jax.ShapeDtypeStruct((B, OUT), jnp.float32),
    )(flat, x32, t0, W1pad, b1, W2, b2)


def kernel(x, table, W1, b1, W2, b2):
    x32 = x.astype(jnp.int32)
    idx = x32.reshape(-1)
    tpad = _transpose_pad(table.T)
    rows = _make_sc_gather()(tpad, idx)
    flat = rows.reshape(B, CTX * P)
    W1pad = jnp.pad(W1.reshape(CTX, E, H), ((0, 0), (0, P - E), (0, 0))).reshape(
        CTX * P, H
    )
    out = _mlp(
        flat, x32, tpad[0:1], W1pad, b1.reshape(1, H), W2, b2.reshape(1, OUT)
    )
    return out


# speedup vs baseline: 2.0207x; 1.1183x over previous
import functools

import jax
import jax.numpy as jnp
from jax import lax
from jax.experimental import pallas as pl
from jax.experimental.pallas import tpu as pltpu
from jax.experimental.pallas import tpu_sc as plsc

B = 16384
V = 1000000
E = 64
CTX = 5
H = 256
OUT = 50

N = B * CTX
P = 2 * E
K = 499712
VP = 500288


@functools.lru_cache(maxsize=1)
def _make_sc_gather():
    info = plsc.get_sparse_core_info()
    NC, NS = info.num_cores, info.num_subcores
    NW = NC * NS
    n_per_w = N // NW
    CH = 320
    n_ch = n_per_w // CH

    mesh = plsc.VectorSubcoreMesh(core_axis_name="c", subcore_axis_name="s")

    @functools.partial(
        pl.kernel,
        mesh=mesh,
        out_type=jax.ShapeDtypeStruct((N, P), jnp.float32),
        scratch_types=[
            pltpu.VMEM((n_per_w,), jnp.int32),
            pltpu.VMEM((CH, P), jnp.float32),
            pltpu.VMEM((CH, P), jnp.float32),
            pltpu.SemaphoreType.DMA,
            pltpu.SemaphoreType.DMA,
        ],
    )
    def gather_k(table_hbm, idx_hbm, out_hbm, idx_v, buf0, buf1, sem0, sem1):
        wid = lax.axis_index("s") * NC + lax.axis_index("c")
        base = wid * n_per_w
        pltpu.sync_copy(idx_hbm.at[pl.ds(base, n_per_w)], idx_v)
        bufs = (buf0, buf1)
        sems = (sem0, sem1)
        copies = [None, None]
        for ci in range(n_ch):
            s = ci % 2
            copies[s] = pltpu.async_copy(
                table_hbm.at[idx_v.at[pl.ds(ci * CH, CH)]], bufs[s], sems[s]
            )
            if ci > 0:
                p = (ci - 1) % 2
                copies[p].wait()
                pltpu.sync_copy(bufs[p], out_hbm.at[pl.ds(base + (ci - 1) * CH, CH)])
        last = (n_ch - 1) % 2
        copies[last].wait()
        pltpu.sync_copy(bufs[last], out_hbm.at[pl.ds(base + (n_ch - 1) * CH, CH)])

    return gather_k


_TPB = 8192


def _tr_body(tl_ref, tr_ref, out_ref):
    ri = lax.broadcasted_iota(jnp.int32, (E, E), 0)
    ci = lax.broadcasted_iota(jnp.int32, (E, E), 1)
    eye = (ri == ci).astype(jnp.float32)
    ttl = lax.dot_general(
        tl_ref[...], eye, (((0,), (0,)), ((), ())),
        preferred_element_type=jnp.float32,
    )
    ttr = lax.dot_general(
        tr_ref[...], eye, (((0,), (0,)), ((), ())),
        preferred_element_type=jnp.float32,
    )
    out_ref[...] = jnp.concatenate([ttl, ttr], axis=1)


def _transpose_pack(tT):
    grid = ((VP + _TPB - 1) // _TPB,)
    return pl.pallas_call(
        _tr_body,
        grid=grid,
        in_specs=[
            pl.BlockSpec((E, _TPB), lambda i: (0, i)),
            pl.BlockSpec((E, _TPB), lambda i: (0, i + K // _TPB)),
        ],
        out_specs=pl.BlockSpec((_TPB, P), lambda i: (i, 0)),
        out_shape=jax.ShapeDtypeStruct((VP, P), jnp.float32),
    )(tT, tT)


_BLK = 2048


def _mlp_body(flat_ref, x_ref, w1d_ref, b1_ref, w2_ref, b2_ref, out_ref):
    x = x_ref[...]
    hi = (x >= K).astype(jnp.float32)
    nz = (x != 0).astype(jnp.float32)
    w_lo = nz * (1.0 - hi)
    padb = jnp.concatenate([w_lo, hi], axis=1)
    ri = lax.broadcasted_iota(jnp.int32, (2 * CTX, CTX * P), 0)
    ci = lax.broadcasted_iota(jnp.int32, (2 * CTX, CTX * P), 1)
    grp = ci // E
    sel = jnp.where(ri < CTX, 2 * ri, 2 * (ri - CTX) + 1)
    S = (grp == sel).astype(jnp.float32)
    mask = jnp.dot(padb, S, preferred_element_type=jnp.float32)
    acc = jnp.dot(
        flat_ref[...] * mask, w1d_ref[...], preferred_element_type=jnp.float32
    )
    h = jnp.tanh(acc + b1_ref[...])
    out_ref[...] = (
        jnp.dot(h, w2_ref[...], preferred_element_type=jnp.float32) + b2_ref[...]
    )


def _mlp(flat, x32, W1dup, b1, W2, b2):
    grid = (B // _BLK,)
    return pl.pallas_call(
        _mlp_body,
        grid=grid,
        in_specs=[
            pl.BlockSpec((_BLK, CTX * P), lambda i: (i, 0)),
            pl.BlockSpec((_BLK, CTX), lambda i: (i, 0)),
            pl.BlockSpec((CTX * P, H), lambda i: (0, 0)),
            pl.BlockSpec((1, H), lambda i: (0, 0)),
            pl.BlockSpec((H, OUT), lambda i: (0, 0)),
            pl.BlockSpec((1, OUT), lambda i: (0, 0)),
        ],
        out_specs=pl.BlockSpec((_BLK, OUT), lambda i: (i, 0)),
        out_shape=jax.ShapeDtypeStruct((B, OUT), jnp.float32),
    )(flat, x32, W1dup, b1, W2, b2)


def kernel(x, table, W1, b1, W2, b2):
    x32 = x.astype(jnp.int32)
    idx = x32.reshape(-1)
    pidx = jnp.where(idx < K, idx, idx - K)
    tp = _transpose_pack(table.T)
    rows = _make_sc_gather()(tp, pidx)
    flat = rows.reshape(B, CTX * P)
    w1r = W1.reshape(CTX, E, H)
    W1dup = jnp.concatenate([w1r, w1r], axis=1).reshape(CTX * P, H)
    out = _mlp(flat, x32, W1dup, b1.reshape(1, H), W2, b2.reshape(1, OUT))
    return out
